# baseline (speedup 1.0000x reference)
import functools

import jax
import jax.numpy as jnp
from jax.experimental import pallas as pl
from jax.experimental.pallas import tpu as pltpu

_DEPTH = 4


def _moe_body(x_hbm, Ws_hbm, bs_ref, Wr_hbm, br_ref, Wg_ref, bg_ref,
              out_ref, logits_ref, xv_ref, wsbuf, wrbuf,
              x_sem, ws_sems, wr_sems):
    E = bs_ref.shape[0]

    def ws_copy(e):
        return pltpu.make_async_copy(Ws_hbm.at[e], wsbuf.at[e % _DEPTH],
                                     ws_sems.at[e % _DEPTH])

    def wr_copy(e):
        return pltpu.make_async_copy(Wr_hbm.at[e], wrbuf.at[e % _DEPTH],
                                     wr_sems.at[e % _DEPTH])

    x_copy = pltpu.make_async_copy(x_hbm, xv_ref, x_sem)
    x_copy.start()
    for e in range(_DEPTH):
        ws_copy(e).start()
        wr_copy(e).start()
    x_copy.wait()

    xbf = xv_ref[...].astype(jnp.bfloat16)
    dot = functools.partial(
        jax.lax.dot_general,
        dimension_numbers=(((1,), (0,)), ((), ())),
        preferred_element_type=jnp.float32)

    logits = dot(xbf, Wg_ref[...].astype(jnp.bfloat16)) + bg_ref[...]
    logits_ref[...] = logits

    iota = jax.lax.broadcasted_iota(jnp.int32, logits.shape, 1)
    m1 = jnp.max(logits, axis=1, keepdims=True)
    a1 = jnp.min(jnp.where(logits == m1, iota, E), axis=1, keepdims=True)
    masked = jnp.where(iota == a1, -jnp.inf, logits)
    m2 = jnp.max(masked, axis=1, keepdims=True)
    a2 = jnp.min(jnp.where(masked == m2, iota, E), axis=1, keepdims=True)
    w1 = 1.0 / (1.0 + jnp.exp(m2 - m1))
    w2 = 1.0 - w1
    comb = (jnp.where(iota == a1, w1, 0.0)
            + jnp.where(iota == a2, w2, 0.0))

    acc = None
    accw = None
    for e in range(E):
        ws_copy(e).wait()
        w = wsbuf[e % _DEPTH]
        accw = w if accw is None else accw + w
        wr_copy(e).wait()
        ye = dot(xbf, wrbuf[e % _DEPTH].astype(jnp.bfloat16))
        yw = (ye * comb[:, e:e + 1]).astype(jnp.bfloat16)
        acc = yw if acc is None else acc + yw
        if e + _DEPTH < E:
            ws_copy(e + _DEPTH).start()
            wr_copy(e + _DEPTH).start()

    shared = dot(xbf, accw.astype(jnp.bfloat16))
    bsum = jnp.sum(bs_ref[...], axis=0, keepdims=True)
    rbias = dot(comb.astype(jnp.bfloat16), br_ref[...].astype(jnp.bfloat16))
    out_ref[...] = acc.astype(jnp.float32) + shared + bsum + rbias


def kernel(x, Ws, bs, Wr, br, Wg, bg):
    b, s, h = x.shape
    E = Ws.shape[0]
    T = b * s
    x2 = x.reshape(T, h)
    bg2 = bg.reshape(1, E)

    out, logits = pl.pallas_call(
        _moe_body,
        in_specs=[
            pl.BlockSpec(memory_space=pltpu.MemorySpace.HBM),
            pl.BlockSpec(memory_space=pltpu.MemorySpace.HBM),
            pl.BlockSpec((E, h), lambda: (0, 0)),
            pl.BlockSpec(memory_space=pltpu.MemorySpace.HBM),
            pl.BlockSpec((E, h), lambda: (0, 0)),
            pl.BlockSpec((h, E), lambda: (0, 0)),
            pl.BlockSpec((1, E), lambda: (0, 0)),
        ],
        out_specs=[
            pl.BlockSpec((T, h), lambda: (0, 0)),
            pl.BlockSpec((T, E), lambda: (0, 0)),
        ],
        out_shape=[
            jax.ShapeDtypeStruct((T, h), jnp.float32),
            jax.ShapeDtypeStruct((T, E), jnp.float32),
        ],
        scratch_shapes=[
            pltpu.VMEM((T, h), jnp.float32),
            pltpu.VMEM((_DEPTH, h, h), jnp.float32),
            pltpu.VMEM((_DEPTH, h, h), jnp.float32),
            pltpu.SemaphoreType.DMA,
            pltpu.SemaphoreType.DMA((_DEPTH,)),
            pltpu.SemaphoreType.DMA((_DEPTH,)),
        ],
    )(x2, Ws, bs, Wr, br, Wg, bg2)

    return out.reshape(b, s, h), logits
